# baseline (speedup 1.0000x reference)
import functools

import jax
import jax.numpy as jnp
from jax import lax
from jax.experimental import pallas as pl
from jax.experimental.pallas import tpu as pltpu
from jax.experimental.pallas import tpu_sc as plsc

_PRIMES = (4993, 4999, 5003, 5009, 5011)
_PMAX = max(_PRIMES)
_ALPHA = 0.0001
_N = 16384
_LANES = 16
_NC, _NS = 2, 16
_HALF = _N // _NC
_GBASE = 2728
_GLEN = 2736
_SLICE = _HALF // _NS
_LN2 = 0.6931471805599453


def _neg_log(x):
    bits = plsc.bitcast(x, jnp.int32)
    e = ((bits >> 23) & 0xFF) - 127
    m = plsc.bitcast((bits & 0x7FFFFF) | 0x3F800000, jnp.float32)
    hi = m >= 1.5
    m = jnp.where(hi, 0.5 * m, m)
    e = jnp.where(hi, e + 1, e)
    s = (m - 1.0) / (m + 1.0)
    s2 = s * s
    poly = 1.0 + s2 * ((1.0 / 3.0) + s2 * ((1.0 / 5.0) + s2 * (1.0 / 7.0)))
    lnm = (2.0 * s) * poly
    return -(e.astype(jnp.float32) * _LN2 + lnm)


def _build():
    scratch = [
        pltpu.VMEM((_GLEN,), jnp.int32),
        pltpu.VMEM((_PMAX,), jnp.float32),
        pltpu.VMEM((_PMAX,), jnp.float32),
        pltpu.VMEM((_GLEN,), jnp.float32),
        pltpu.VMEM((1,), jnp.float32),
        pltpu.VMEM((5 * _SLICE,), jnp.float32),
        pltpu.VMEM((_SLICE,), jnp.float32),
        pltpu.VMEM_SHARED((5 * _HALF,), jnp.float32),
        pltpu.SemaphoreType.DMA,
        pltpu.SemaphoreType.DMA,
    ]
    mesh = plsc.VectorSubcoreMesh(core_axis_name="c", subcore_axis_name="s")

    @functools.partial(
        pl.kernel,
        out_type=jax.ShapeDtypeStruct((_N,), jnp.float32),
        mesh=mesh,
        scratch_types=scratch,
        compiler_params=pltpu.CompilerParams(needs_layout_passes=False),
    )
    def sc_kernel(items_h, a0h, a1h, a2h, a3h, a4h, b0h, b1h, b2h, b3h, b4h,
                  t_h, out_h,
                  it_v, a_v, b_v, d_v, t_v, col_v, out_v, shared, sem0, sem1):
        c = lax.axis_index("c")
        s = lax.axis_index("s")
        ti = s % 5
        g = jnp.minimum(s // 5, 2)
        gbase = c * _HALF + g * _GBASE
        c_it = pltpu.async_copy(items_h.at[pl.ds(gbase, _GLEN)], it_v, sem0)
        c_t = pltpu.async_copy(t_h, t_v, sem0)
        for i, (ah, bh) in enumerate(
                zip((a0h, a1h, a2h, a3h, a4h), (b0h, b1h, b2h, b3h, b4h))):
            @pl.when(ti == i)
            def _(ah=ah, bh=bh, p=_PRIMES[i]):
                ca = pltpu.async_copy(ah, a_v.at[pl.ds(0, p)], sem1)
                cb = pltpu.async_copy(bh, b_v.at[pl.ds(0, p)], sem1)
                ca.wait()
                cb.wait()
        p_s = jnp.int32(_PRIMES[0])
        inv_s = jnp.float32(1.0 / _PRIMES[0])
        for i, p in enumerate(_PRIMES[1:], start=1):
            p_s = jnp.where(ti == i, jnp.int32(p), p_s)
            inv_s = jnp.where(ti == i, jnp.float32(1.0 / p), inv_s)
        c_it.wait()
        c_t.wait()

        @plsc.parallel_loop(0, _GLEN, step=_LANES, unroll=1)
        def _phase1(off):
            it = it_v[pl.ds(off, _LANES)]
            q = (it.astype(jnp.float32) * inv_s).astype(jnp.int32)
            r = it - q * p_s
            r = jnp.where(r < 0, r + p_s, r)
            r = jnp.where(r >= p_s, r - p_s, r)
            a = plsc.load_gather(a_v, [r])
            b = plsc.load_gather(b_v, [r])
            d_v[pl.ds(off, _LANES)] = (1.0 - _ALPHA) * b - _ALPHA * a

        pltpu.async_copy(
            d_v, shared.at[pl.ds(ti * _HALF + g * _GBASE, _GLEN)], sem0).wait()
        plsc.subcore_barrier()

        sbase = s * _SLICE
        ccols = [pltpu.async_copy(
            shared.at[pl.ds(i * _HALF + sbase, _SLICE)],
            col_v.at[pl.ds(i * _SLICE, _SLICE)], sem0) for i in range(5)]
        for cc in ccols:
            cc.wait()
        z16 = jnp.zeros((_LANES,), jnp.int32)
        cvec = (plsc.load_gather(t_v, [z16]) + 1.0) * _ALPHA

        @plsc.parallel_loop(0, _SLICE, step=_LANES, unroll=1)
        def _phase2(off):
            best = col_v[pl.ds(off, _LANES)]
            for i in range(1, 5):
                best = jnp.maximum(
                    best, col_v[pl.ds(i * _SLICE + off, _LANES)])
            out_v[pl.ds(off, _LANES)] = _neg_log(best + cvec)

        pltpu.async_copy(
            out_v, out_h.at[pl.ds(c * _HALF + sbase, _SLICE)], sem0).wait()

    return sc_kernel


_SC_KERNEL = _build()


def kernel(items, A0, A1, A2, A3, A4, B0, B1, B2, B3, B4, t):
    return _SC_KERNEL(items, A0, A1, A2, A3, A4, B0, B1, B2, B3, B4, t)


# speedup vs baseline: 1.0082x; 1.0082x over previous
import functools

import jax
import jax.numpy as jnp
from jax import lax
from jax.experimental import pallas as pl
from jax.experimental.pallas import tpu as pltpu
from jax.experimental.pallas import tpu_sc as plsc

_PRIMES = (4993, 4999, 5003, 5009, 5011)
_PMAX = max(_PRIMES)
_ALPHA = 0.0001
_N = 16384
_LANES = 16
_NC, _NS = 2, 16
_HALF = _N // _NC
_GBASE = 2728
_GLEN = 2736
_SLICE = _HALF // _NS
_LN2 = 0.6931471805599453


def _neg_log(x):
    bits = plsc.bitcast(x, jnp.int32)
    e = ((bits >> 23) & 0xFF) - 127
    m = plsc.bitcast((bits & 0x7FFFFF) | 0x3F800000, jnp.float32)
    hi = m >= 1.5
    m = jnp.where(hi, 0.5 * m, m)
    e = jnp.where(hi, e + 1, e)
    s = (m - 1.0) / (m + 1.0)
    s2 = s * s
    poly = 1.0 + s2 * ((1.0 / 3.0) + s2 * ((1.0 / 5.0) + s2 * (1.0 / 7.0)))
    lnm = (2.0 * s) * poly
    return -(e.astype(jnp.float32) * _LN2 + lnm)


def _build():
    scratch = [
        pltpu.VMEM((_GLEN,), jnp.int32),
        pltpu.VMEM((_PMAX,), jnp.float32),
        pltpu.VMEM((_PMAX,), jnp.float32),
        pltpu.VMEM((_GLEN,), jnp.float32),
        pltpu.VMEM((1,), jnp.float32),
        pltpu.VMEM((5 * _SLICE,), jnp.float32),
        pltpu.VMEM((_SLICE,), jnp.float32),
        pltpu.VMEM_SHARED((5 * _HALF,), jnp.float32),
        pltpu.SemaphoreType.DMA,
        pltpu.SemaphoreType.DMA,
    ]
    mesh = plsc.VectorSubcoreMesh(core_axis_name="c", subcore_axis_name="s")

    @functools.partial(
        pl.kernel,
        out_type=jax.ShapeDtypeStruct((_N,), jnp.float32),
        mesh=mesh,
        scratch_types=scratch,
        compiler_params=pltpu.CompilerParams(needs_layout_passes=False),
    )
    def sc_kernel(items_h, a0h, a1h, a2h, a3h, a4h, b0h, b1h, b2h, b3h, b4h,
                  t_h, out_h,
                  it_v, a_v, b_v, d_v, t_v, col_v, out_v, shared, sem0, sem1):
        c = lax.axis_index("c")
        s = lax.axis_index("s")
        ti = s % 5
        g = jnp.minimum(s // 5, 2)
        gbase = c * _HALF + g * _GBASE
        c_it = pltpu.async_copy(items_h.at[pl.ds(gbase, _GLEN)], it_v, sem0)
        c_t = pltpu.async_copy(t_h, t_v, sem0)
        for i, (ah, bh) in enumerate(
                zip((a0h, a1h, a2h, a3h, a4h), (b0h, b1h, b2h, b3h, b4h))):
            @pl.when(ti == i)
            def _(ah=ah, bh=bh, p=_PRIMES[i]):
                ca = pltpu.async_copy(ah, a_v.at[pl.ds(0, p)], sem1)
                cb = pltpu.async_copy(bh, b_v.at[pl.ds(0, p)], sem1)
                ca.wait()
                cb.wait()
        p_s = jnp.int32(_PRIMES[0])
        inv_s = jnp.float32(1.0 / _PRIMES[0])
        for i, p in enumerate(_PRIMES[1:], start=1):
            p_s = jnp.where(ti == i, jnp.int32(p), p_s)
            inv_s = jnp.where(ti == i, jnp.float32(1.0 / p), inv_s)
        c_it.wait()
        c_t.wait()

        @plsc.parallel_loop(0, _GLEN, step=_LANES, unroll=2)
        def _phase1(off):
            it = it_v[pl.ds(off, _LANES)]
            q = (it.astype(jnp.float32) * inv_s).astype(jnp.int32)
            r = it - q * p_s
            r = jnp.where(r < 0, r + p_s, r)
            r = jnp.where(r >= p_s, r - p_s, r)
            a = plsc.load_gather(a_v, [r])
            b = plsc.load_gather(b_v, [r])
            d_v[pl.ds(off, _LANES)] = (1.0 - _ALPHA) * b - _ALPHA * a

        pltpu.async_copy(
            d_v, shared.at[pl.ds(ti * _HALF + g * _GBASE, _GLEN)], sem0).wait()
        plsc.subcore_barrier()

        sbase = s * _SLICE
        ccols = [pltpu.async_copy(
            shared.at[pl.ds(i * _HALF + sbase, _SLICE)],
            col_v.at[pl.ds(i * _SLICE, _SLICE)], sem0) for i in range(5)]
        for cc in ccols:
            cc.wait()
        z16 = jnp.zeros((_LANES,), jnp.int32)
        cvec = (plsc.load_gather(t_v, [z16]) + 1.0) * _ALPHA

        @plsc.parallel_loop(0, _SLICE, step=_LANES, unroll=2)
        def _phase2(off):
            best = col_v[pl.ds(off, _LANES)]
            for i in range(1, 5):
                best = jnp.maximum(
                    best, col_v[pl.ds(i * _SLICE + off, _LANES)])
            out_v[pl.ds(off, _LANES)] = _neg_log(best + cvec)

        pltpu.async_copy(
            out_v, out_h.at[pl.ds(c * _HALF + sbase, _SLICE)], sem0).wait()

    return sc_kernel


_SC_KERNEL = _build()


def kernel(items, A0, A1, A2, A3, A4, B0, B1, B2, B3, B4, t):
    return _SC_KERNEL(items, A0, A1, A2, A3, A4, B0, B1, B2, B3, B4, t)
